# baseline (speedup 1.0000x reference)
import jax
import jax.numpy as jnp
from jax import lax
from jax.experimental import pallas as pl
from jax.experimental.pallas import tpu as pltpu
from jax.experimental.pallas import tpu_sc as plsc

D = 64
B0, B1 = 4096, 200
NPAIR = 500_000
NW = 32
N_UNITS_TOTAL = B1 * (B0 // 128)
PER_W = N_UNITS_TOTAL // NW
NSLOT = 4

_mesh = plsc.VectorSubcoreMesh(core_axis_name="core", subcore_axis_name="subcore")


def _gather(tableP, xT):
    @pl.kernel(
        out_type=jax.ShapeDtypeStruct((B1, D, B0), jnp.float32),
        mesh=_mesh,
        compiler_params=pltpu.CompilerParams(needs_layout_passes=False),
        scratch_types=[
            pltpu.VMEM((128,), jnp.int32),
            pltpu.VMEM((128,), jnp.int32),
            pltpu.VMEM((128,), jnp.int32),
            pltpu.VMEM((128,), jnp.int32),
            pltpu.VMEM((128,), jnp.int32),
            pltpu.VMEM((128,), jnp.int32),
            pltpu.VMEM((128,), jnp.int32),
            pltpu.VMEM((128,), jnp.int32),
            pltpu.VMEM((128,), jnp.int32),
            pltpu.VMEM((128, 128), jnp.float32),
            pltpu.VMEM((128, 128), jnp.float32),
            pltpu.VMEM((128, 128), jnp.float32),
            pltpu.VMEM((128, 128), jnp.float32),
            pltpu.VMEM((D, 128), jnp.float32),
            pltpu.VMEM((D, 128), jnp.float32),
            pltpu.VMEM((D, 128), jnp.float32),
            pltpu.VMEM((D, 128), jnp.float32),
            pltpu.SemaphoreType.DMA,
            pltpu.SemaphoreType.DMA,
            pltpu.SemaphoreType.DMA,
            pltpu.SemaphoreType.DMA,
            pltpu.SemaphoreType.DMA,
            pltpu.SemaphoreType.DMA,
            pltpu.SemaphoreType.DMA,
            pltpu.SemaphoreType.DMA,
        ],
    )
    def k(tab, xt, out, ib, q0, q1, q2, q3, a0, a1, a2, a3, g0, g1, g2, g3,
          o0, o1, o2, o3, sg0, sg1, sg2, sg3, so0, so1, so2, so3):
        qbufs = (q0, q1, q2, q3)
        w = lax.axis_index("subcore") * 2 + lax.axis_index("core")
        abufs = (a0, a1, a2, a3)
        gbufs = (g0, g1, g2, g3)
        obufs = (o0, o1, o2, o3)
        sgs = (sg0, sg1, sg2, sg3)
        sos = (so0, so1, so2, so3)
        iota = lax.iota(jnp.int32, 16)

        def unit_pos(u):
            uu = w * PER_W + u
            return uu // 32, (uu % 32) * 128

        def gather_copy(b):
            return pltpu.make_async_copy(tab.at[qbufs[b]], gbufs[b], sgs[b])

        def write_copy(u, b):
            j, i0 = unit_pos(u)
            return pltpu.make_async_copy(
                obufs[b], out.at[j, :, pl.ds(i0, 128)], sos[b])

        def prep(u, b):
            j, i0 = unit_pos(u)
            pltpu.sync_copy(xt.at[j, pl.ds(i0, 128)], ib)
            for kk in range(0, 128, 16):
                iv = ib[pl.ds(kk, 16)]
                qbufs[b][pl.ds(kk, 16)] = lax.shift_right_logical(iv, 1)
                abufs[b][pl.ds(kk, 16)] = (iv & 1) * D
            gather_copy(b).start()

        def consume(u, b):
            gather_copy(b).wait()
            gb, ab, ob = gbufs[b], abufs[b], obufs[b]
            rows = [iota + lg for lg in range(0, 128, 16)]
            cols = [ab[pl.ds(lg, 16)] for lg in range(0, 128, 16)]

            @pl.loop(0, D)
            def _(d):
                for gidx in range(8):
                    v = plsc.load_gather(gb, [rows[gidx], cols[gidx] + d])
                    ob[d, pl.ds(gidx * 16, 16)] = v

            write_copy(u, b).start()

        prep(0, 0)
        prep(1, 1)

        @pl.loop(0, PER_W, step=NSLOT)
        def _(u0):
            for db in range(NSLOT):
                u = u0 + db
                b = db % NSLOT

                @pl.when(u >= NSLOT)
                def _():
                    write_copy(u - NSLOT, b).wait()

                @pl.when(u + 2 < PER_W)
                def _():
                    prep(u + 2, (db + 2) % NSLOT)

                consume(u, b)

        for uu in range(PER_W - NSLOT, PER_W):
            write_copy(uu, uu % NSLOT).wait()

    return k(tableP, xT)


def kernel(x, table):
    xT = x.T.astype(jnp.int32)
    tableP = table.reshape(NPAIR, 128)
    outT = _gather(tableP, xT)
    return outT.transpose(2, 0, 1)


# speedup vs baseline: 1.0464x; 1.0464x over previous
import jax
import jax.numpy as jnp
from jax import lax
from jax.experimental import pallas as pl
from jax.experimental.pallas import tpu as pltpu
from jax.experimental.pallas import tpu_sc as plsc

D = 64
B0, B1 = 4096, 200
NPAIR = 500_000
NSLOT = 4

_mesh = plsc.VectorSubcoreMesh(core_axis_name="core", subcore_axis_name="subcore")


def _gather(tableP, xT):
    @pl.kernel(
        out_type=jax.ShapeDtypeStruct((B1, D, B0), jnp.float32),
        mesh=_mesh,
        compiler_params=pltpu.CompilerParams(needs_layout_passes=False),
        scratch_types=[
            pltpu.VMEM((B1, 128), jnp.int32),
            pltpu.VMEM((128,), jnp.int32),
            pltpu.VMEM((128,), jnp.int32),
            pltpu.VMEM((128,), jnp.int32),
            pltpu.VMEM((128,), jnp.int32),
            pltpu.VMEM((128,), jnp.int32),
            pltpu.VMEM((128,), jnp.int32),
            pltpu.VMEM((128,), jnp.int32),
            pltpu.VMEM((128,), jnp.int32),
            pltpu.VMEM((128, 128), jnp.float32),
            pltpu.VMEM((128, 128), jnp.float32),
            pltpu.VMEM((128, 128), jnp.float32),
            pltpu.VMEM((128, 128), jnp.float32),
            pltpu.VMEM((D, 128), jnp.float32),
            pltpu.VMEM((D, 128), jnp.float32),
            pltpu.SemaphoreType.DMA,
            pltpu.SemaphoreType.DMA,
            pltpu.SemaphoreType.DMA,
            pltpu.SemaphoreType.DMA,
            pltpu.SemaphoreType.DMA,
            pltpu.SemaphoreType.DMA,
        ],
    )
    def k(tab, xt, out, xtb, q0, q1, q2, q3, a0, a1, a2, a3,
          g0, g1, g2, g3, o0, o1, sg0, sg1, sg2, sg3, so0, so1):
        w = lax.axis_index("subcore") * 2 + lax.axis_index("core")
        qbufs = (q0, q1, q2, q3)
        abufs = (a0, a1, a2, a3)
        gbufs = (g0, g1, g2, g3)
        obufs = (o0, o1)
        sgs = (sg0, sg1, sg2, sg3)
        sos = (so0, so1)
        iota = lax.iota(jnp.int32, 16)
        i0 = w * 128

        pltpu.sync_copy(xt.at[:, pl.ds(i0, 128)], xtb)

        def gather_copy(b):
            return pltpu.make_async_copy(tab.at[qbufs[b]], gbufs[b], sgs[b])

        def write_copy(j, b2):
            return pltpu.make_async_copy(
                obufs[b2], out.at[j, :, pl.ds(i0, 128)], sos[b2])

        def prep(j, b):
            for kk in range(0, 128, 16):
                iv = xtb[j, pl.ds(kk, 16)]
                qbufs[b][pl.ds(kk, 16)] = lax.shift_right_logical(iv, 1)
                abufs[b][pl.ds(kk, 16)] = (iv & 1) * D
            gather_copy(b).start()

        def consume(j, b, b2):
            gather_copy(b).wait()
            gb, ab, ob = gbufs[b], abufs[b], obufs[b2]
            rows = [iota + lg for lg in range(0, 128, 16)]
            cols = [ab[pl.ds(lg, 16)] for lg in range(0, 128, 16)]

            @pl.loop(0, D)
            def _(d):
                for gidx in range(8):
                    v = plsc.load_gather(gb, [rows[gidx], cols[gidx] + d])
                    ob[d, pl.ds(gidx * 16, 16)] = v

            write_copy(j, b2).start()

        prep(0, 0)
        prep(1, 1)

        @pl.loop(0, B1, step=NSLOT)
        def _(j0):
            for db in range(NSLOT):
                j = j0 + db
                b = db % NSLOT
                b2 = db % 2

                @pl.when(j >= 2)
                def _():
                    write_copy(j - 2, b2).wait()

                @pl.when(j + 2 < B1)
                def _():
                    prep(j + 2, (db + 2) % NSLOT)

                consume(j, b, b2)

        write_copy(B1 - 2, 0).wait()
        write_copy(B1 - 1, 1).wait()

    return k(tableP, xT)


def kernel(x, table):
    xT = x.T.astype(jnp.int32)
    tableP = table.reshape(NPAIR, 128)
    outT = _gather(tableP, xT)
    return outT.transpose(2, 0, 1)


# speedup vs baseline: 1.5583x; 1.4892x over previous
import jax
import jax.numpy as jnp
from jax import lax
from jax.experimental import pallas as pl
from jax.experimental.pallas import tpu as pltpu
from jax.experimental.pallas import tpu_sc as plsc

D = 64
B0, B1 = 4096, 200
NW = 32
PER_W = B0 // NW * B1
UNIT = 256
N_UNITS = PER_W // UNIT
NSLOT = 4

_mesh = plsc.VectorSubcoreMesh(core_axis_name="core", subcore_axis_name="subcore")


def _flatten_idx(xT):
    @pl.kernel(
        out_type=jax.ShapeDtypeStruct((B0 * B1,), jnp.int32),
        mesh=_mesh,
        compiler_params=pltpu.CompilerParams(needs_layout_passes=False),
        scratch_types=[
            pltpu.VMEM((B1, 128), jnp.int32),
            pltpu.VMEM((PER_W,), jnp.int32),
        ],
    )
    def k(xt, xfl, xtb, xfb):
        w = lax.axis_index("subcore") * 2 + lax.axis_index("core")
        iota = lax.iota(jnp.int32, 16)
        i0 = w * 128
        pltpu.sync_copy(xt.at[:, pl.ds(i0, 128)], xtb)
        j_chunks = list(range(0, B1 - 16, 16)) + [B1 - 16]

        @pl.loop(0, 128)
        def _(ii):
            ci = iota * 0 + ii
            for j0 in j_chunks:
                v = plsc.load_gather(xtb, [iota + j0, ci])
                xfb[pl.ds(ii * B1 + j0, 16)] = v

        pltpu.sync_copy(xfb, xfl.at[pl.ds(w * PER_W, PER_W)])

    return k(xT)


def _gather(table, xfl):
    @pl.kernel(
        out_type=jax.ShapeDtypeStruct((B0 * B1, D), jnp.float32),
        mesh=_mesh,
        compiler_params=pltpu.CompilerParams(
            use_tc_tiling_on_sc=False, needs_layout_passes=False),
        scratch_types=[
            pltpu.VMEM((PER_W,), jnp.int32),
            pltpu.VMEM((UNIT, D), jnp.float32),
            pltpu.VMEM((UNIT, D), jnp.float32),
            pltpu.VMEM((UNIT, D), jnp.float32),
            pltpu.VMEM((UNIT, D), jnp.float32),
            pltpu.SemaphoreType.DMA,
            pltpu.SemaphoreType.DMA,
            pltpu.SemaphoreType.DMA,
            pltpu.SemaphoreType.DMA,
            pltpu.SemaphoreType.DMA,
            pltpu.SemaphoreType.DMA,
            pltpu.SemaphoreType.DMA,
            pltpu.SemaphoreType.DMA,
        ],
    )
    def k(tab, xf, out, xfb, g0, g1, g2, g3,
          sg0, sg1, sg2, sg3, so0, so1, so2, so3):
        w = lax.axis_index("subcore") * 2 + lax.axis_index("core")
        gbufs = (g0, g1, g2, g3)
        sgs = (sg0, sg1, sg2, sg3)
        sos = (so0, so1, so2, so3)
        base = w * PER_W

        pltpu.sync_copy(xf.at[pl.ds(base, PER_W)], xfb)

        def gather_copy(u, b):
            return pltpu.make_async_copy(
                tab.at[xfb.at[pl.ds(u * UNIT, UNIT)]], gbufs[b], sgs[b])

        def write_copy(u, b):
            return pltpu.make_async_copy(
                gbufs[b], out.at[pl.ds(base + u * UNIT, UNIT), :], sos[b])

        gather_copy(0, 0).start()
        gather_copy(1, 1).start()

        @pl.loop(0, N_UNITS, step=NSLOT)
        def _(u0):
            for db in range(NSLOT):
                u = u0 + db
                b = db % NSLOT
                nb = (db + 2) % NSLOT

                @pl.when(u >= 2)
                def _():
                    write_copy(u - 2, nb).wait()

                @pl.when(u + 2 < N_UNITS)
                def _():
                    gather_copy(u + 2, nb).start()

                gather_copy(u, b).wait()
                write_copy(u, b).start()

        write_copy(N_UNITS - 2, (N_UNITS - 2) % NSLOT).wait()
        write_copy(N_UNITS - 1, (N_UNITS - 1) % NSLOT).wait()

    return k(table, xfl)


def kernel(x, table):
    xT = x.T.astype(jnp.int32)
    xfl = _flatten_idx(xT)
    out = _gather(table, xfl)
    return out.reshape(B0, B1, D)
